# Initial kernel scaffold; baseline (speedup 1.0000x reference)
import jax
import jax.numpy as jnp
from jax.experimental import pallas as pl


def kernel(batch_feat, batch_edges, batch_attr, W1, b1, W2, b2, W22, b22, W3, b3):
    raise NotImplementedError("write your pallas kernel here")


# speedup vs baseline: 33.4786x; 33.4786x over previous
import functools
import jax
import jax.numpy as jnp
from jax import lax
from jax.experimental import pallas as pl
from jax.experimental.pallas import tpu as pltpu
from jax.experimental.pallas import tpu_sc as plsc

B = 4
N = 10000
NP = 10240
NR = NP // 128
E = 320000
EP = 327680
ER = EP // 128
F_IN = 128
D = 64

NC = 2
NS = 16
EPT = EP // NS
NBLK = EPT // 2048

_mesh = plsc.VectorSubcoreMesh(core_axis_name="c", subcore_axis_name="s")


NRED = NP // 10


def _zero_flat(ref, nwords):
    def body(i, _):
        ref[pl.ds(16 * i, 16)] = jnp.zeros((16,), jnp.float32)
        return 0
    lax.fori_loop(0, nwords // 16, body, 0)


def _slab_reduce_out(acc, slab, tmp, red, sid, out_slice):
    pltpu.sync_copy(acc, slab.at[sid])
    plsc.subcore_barrier()

    @pl.when(sid < 10)
    def _():
        pltpu.sync_copy(slab.at[0, pl.ds(NRED * sid, NRED)], red)

        def addk(k, _):
            pltpu.sync_copy(slab.at[k, pl.ds(NRED * sid, NRED)], tmp)

            def vadd(i, _):
                red[pl.ds(16 * i, 16)] = (red[pl.ds(16 * i, 16)]
                                          + tmp[pl.ds(16 * i, 16)])
                return 0
            lax.fori_loop(0, NRED // 16, vadd, 0)
            return 0
        lax.fori_loop(1, NS, addk, 0)
        out_slice(sid, red)
    plsc.subcore_barrier()


@functools.partial(
    pl.kernel,
    out_type=jax.ShapeDtypeStruct((B, 1, NP), jnp.float32),
    mesh=_mesh,
    compiler_params=pltpu.CompilerParams(needs_layout_passes=False, use_tc_tiling_on_sc=False),
    scratch_types=[
        pltpu.VMEM((16, 128), jnp.int32),
        pltpu.VMEM((16, 128), jnp.float32),
        pltpu.VMEM((NP,), jnp.float32),
        pltpu.VMEM((NRED,), jnp.float32),
        pltpu.VMEM((NRED,), jnp.float32),
        pltpu.VMEM_SHARED((NS, NP), jnp.float32),
    ],
)
def _deg_kernel(dst_hbm, ew_hbm, deg_hbm, dbuf, wbuf, acc, tmp, red, slab):
    cid = lax.axis_index("c")
    sid = lax.axis_index("s")
    row0 = sid * (EPT // 128)

    def per_graph(gi, _):
        b = 2 * cid + gi
        _zero_flat(acc, NP)

        def per_block(k, _):
            pltpu.sync_copy(dst_hbm.at[b, pl.ds(row0 + 16 * k, 16)], dbuf)
            pltpu.sync_copy(ew_hbm.at[b, pl.ds(row0 + 16 * k, 16)], wbuf)

            def per_row(j, _):
                for u in range(8):
                    dvec = dbuf[j, pl.ds(16 * u, 16)]
                    wvec = wbuf[j, pl.ds(16 * u, 16)]
                    plsc.addupdate_scatter(acc, [dvec], wvec)
                return 0
            lax.fori_loop(0, 16, per_row, 0)
            return 0
        lax.fori_loop(0, NBLK, per_block, 0)

        def out_slice(t, red_ref):
            pltpu.sync_copy(red_ref, deg_hbm.at[b, 0, pl.ds(NRED * t, NRED)])
        _slab_reduce_out(acc, slab, tmp, red, sid, out_slice)
        return 0
    lax.fori_loop(0, B // NC, per_graph, 0)


@functools.partial(
    pl.kernel,
    out_type=jax.ShapeDtypeStruct((B, NP, D), jnp.float32),
    mesh=_mesh,
    compiler_params=pltpu.CompilerParams(needs_layout_passes=False, use_tc_tiling_on_sc=False),
    scratch_types=[
        pltpu.VMEM((16, 128), jnp.int32),
        pltpu.VMEM((16, 128), jnp.int32),
        pltpu.VMEM((16, 128), jnp.float32),
        pltpu.VMEM((128, D), jnp.float32),
        pltpu.VMEM((64, D), jnp.float32),
        pltpu.SemaphoreType.DMA,
        pltpu.VMEM_SHARED((NP, D), jnp.float32),
    ],
)
def _spmm_kernel(g_hbm, src_hbm, dst_hbm, ew_hbm, s_hbm,
                 sbuf, dbuf, wbuf, rows, zbuf, sem, out_sh):
    cid = lax.axis_index("c")
    sid = lax.axis_index("s")
    row0 = sid * (EPT // 128)
    nslice = NP // NS
    nd0 = sid * nslice

    def zrow(r, _):
        for u in range(D // 16):
            zbuf[r, pl.ds(16 * u, 16)] = jnp.zeros((16,), jnp.float32)
        return 0
    lax.fori_loop(0, 64, zrow, 0)

    def per_graph(gi, _):
        b = 2 * cid + gi
        def zcp(m, _):
            pltpu.sync_copy(zbuf, out_sh.at[pl.ds(nd0 + 64 * m, 64)])
            return 0
        lax.fori_loop(0, nslice // 64, zcp, 0)
        plsc.subcore_barrier()

        def per_block(k, _):
            pltpu.sync_copy(src_hbm.at[b, pl.ds(row0 + 16 * k, 16)], sbuf)
            pltpu.sync_copy(dst_hbm.at[b, pl.ds(row0 + 16 * k, 16)], dbuf)
            pltpu.sync_copy(ew_hbm.at[b, pl.ds(row0 + 16 * k, 16)], wbuf)

            def per_chunk(j, _):
                pltpu.async_copy(g_hbm.at[b].at[sbuf.at[j]], rows, sem).wait()

                def scale(e, _):
                    w = plsc.load_gather(
                        wbuf, [jnp.full((16,), j, jnp.int32),
                               jnp.full((16,), e, jnp.int32)])
                    for u in range(D // 16):
                        rows[e, pl.ds(16 * u, 16)] = rows[e, pl.ds(16 * u, 16)] * w
                    return 0
                lax.fori_loop(0, 128, scale, 0)
                pltpu.sync_copy(rows, out_sh.at[dbuf.at[j]], add=True)
                return 0
            lax.fori_loop(0, 16, per_chunk, 0)
            return 0
        lax.fori_loop(0, NBLK, per_block, 0)

        plsc.subcore_barrier()
        pltpu.sync_copy(out_sh.at[pl.ds(nd0, nslice)],
                        s_hbm.at[b, pl.ds(nd0, nslice)])
        plsc.subcore_barrier()
        return 0
    lax.fori_loop(0, B // NC, per_graph, 0)


@functools.partial(
    pl.kernel,
    out_type=jax.ShapeDtypeStruct((B, 1, NP), jnp.float32),
    mesh=_mesh,
    compiler_params=pltpu.CompilerParams(needs_layout_passes=False, use_tc_tiling_on_sc=False),
    scratch_types=[
        pltpu.VMEM((16, 128), jnp.int32),
        pltpu.VMEM((16, 128), jnp.int32),
        pltpu.VMEM((16, 128), jnp.float32),
        pltpu.VMEM((NP,), jnp.float32),
        pltpu.VMEM((NP,), jnp.float32),
        pltpu.VMEM((NRED,), jnp.float32),
        pltpu.VMEM((NRED,), jnp.float32),
        pltpu.VMEM_SHARED((NS, NP), jnp.float32),
    ],
)
def _spmv_kernel(g_hbm, src_hbm, dst_hbm, ew_hbm, s_hbm,
                 sbuf, dbuf, wbuf, gvec, acc, tmp, red, slab):
    cid = lax.axis_index("c")
    sid = lax.axis_index("s")
    row0 = sid * (EPT // 128)

    def per_graph(gi, _):
        b = 2 * cid + gi
        pltpu.sync_copy(g_hbm.at[b, 0], gvec)
        _zero_flat(acc, NP)

        def per_block(k, _):
            pltpu.sync_copy(src_hbm.at[b, pl.ds(row0 + 16 * k, 16)], sbuf)
            pltpu.sync_copy(dst_hbm.at[b, pl.ds(row0 + 16 * k, 16)], dbuf)
            pltpu.sync_copy(ew_hbm.at[b, pl.ds(row0 + 16 * k, 16)], wbuf)

            def per_row(j, _):
                for u in range(8):
                    svec = sbuf[j, pl.ds(16 * u, 16)]
                    dvec = dbuf[j, pl.ds(16 * u, 16)]
                    wvec = wbuf[j, pl.ds(16 * u, 16)]
                    vals = plsc.load_gather(gvec, [svec])
                    plsc.addupdate_scatter(acc, [dvec], vals * wvec)
                return 0
            lax.fori_loop(0, 16, per_row, 0)
            return 0
        lax.fori_loop(0, NBLK, per_block, 0)

        def out_slice(t, red_ref):
            pltpu.sync_copy(red_ref, s_hbm.at[b, 0, pl.ds(NRED * t, NRED)])
        _slab_reduce_out(acc, slab, tmp, red, sid, out_slice)
        return 0
    lax.fori_loop(0, B // NC, per_graph, 0)


def _dis(deg):
    return jnp.where(deg > 0, lax.rsqrt(jnp.where(deg > 0, deg, 1.0)), 0.0)


def _k1_body(deg_ref, x_ref, w_ref, o_ref):
    dis = _dis(deg_ref[0])
    o_ref[0] = jnp.dot(x_ref[0], w_ref[...],
                       preferred_element_type=jnp.float32) * dis


def _kmid_body(deg_ref, s_ref, b_ref, w_ref, o_ref):
    dis = _dis(deg_ref[0])
    x = jnp.maximum(s_ref[0] * dis + b_ref[...], 0.0)
    o_ref[0] = jnp.dot(x, w_ref[...],
                       preferred_element_type=jnp.float32) * dis


def _kfinal_body(deg_ref, s_ref, b_ref, o_ref):
    deg = deg_ref[0]
    o = s_ref[0] * _dis(deg) + b_ref[...]
    idx = (lax.broadcasted_iota(jnp.int32, (NR, 128), 0) * 128
           + lax.broadcasted_iota(jnp.int32, (NR, 128), 1))
    mask = (idx >= 1) & (idx < N - 1)
    m = jnp.max(jnp.where(mask, o, -jnp.inf))
    ex = jnp.where(mask, jnp.exp(o - m), 0.0)
    o_ref[0] = ex / jnp.sum(ex)


_RB = 1280
_NRB = NP // _RB


def _tc_matmul1(deg_c, xp, W):
    return pl.pallas_call(
        _k1_body,
        grid=(B, _NRB),
        in_specs=[
            pl.BlockSpec((1, _RB, 1), lambda b, i: (b, i, 0)),
            pl.BlockSpec((1, _RB, F_IN), lambda b, i: (b, i, 0)),
            pl.BlockSpec((F_IN, D), lambda b, i: (0, 0)),
        ],
        out_specs=pl.BlockSpec((1, _RB, D), lambda b, i: (b, i, 0)),
        out_shape=jax.ShapeDtypeStruct((B, NP, D), jnp.float32),
    )(deg_c, xp, W)


def _tc_matmul_mid(deg_c, s, bias, W):
    wo = W.shape[1]
    return pl.pallas_call(
        _kmid_body,
        grid=(B, _NRB),
        in_specs=[
            pl.BlockSpec((1, _RB, 1), lambda b, i: (b, i, 0)),
            pl.BlockSpec((1, _RB, D), lambda b, i: (b, i, 0)),
            pl.BlockSpec((D,), lambda b, i: (0,)),
            pl.BlockSpec((D, wo), lambda b, i: (0, 0)),
        ],
        out_specs=pl.BlockSpec((1, _RB, wo), lambda b, i: (b, i, 0)),
        out_shape=jax.ShapeDtypeStruct((B, NP, wo), jnp.float32),
    )(deg_c, s, bias, W)


def _tc_final(deg_r, s4, b3):
    return pl.pallas_call(
        _kfinal_body,
        grid=(B,),
        in_specs=[
            pl.BlockSpec((1, NR, 128), lambda b: (b, 0, 0)),
            pl.BlockSpec((1, NR, 128), lambda b: (b, 0, 0)),
            pl.BlockSpec((1,), lambda b: (0,)),
        ],
        out_specs=pl.BlockSpec((1, NR, 128), lambda b: (b, 0, 0)),
        out_shape=jax.ShapeDtypeStruct((B, NR, 128), jnp.float32),
    )(deg_r, s4, b3)


def kernel(batch_feat, batch_edges, batch_attr, W1, b1, W2, b2, W22, b22, W3, b3):
    xp = jnp.pad(batch_feat, ((0, 0), (0, NP - N), (0, 0)))
    src = jnp.pad(batch_edges[:, 0, :], ((0, 0), (0, EP - E)))
    dst = jnp.pad(batch_edges[:, 1, :], ((0, 0), (0, EP - E)))
    ew = jnp.pad(batch_attr, ((0, 0), (0, EP - E)))
    src3 = src.reshape(B, ER, 128)
    dst3 = dst.reshape(B, ER, 128)
    ew3 = ew.reshape(B, ER, 128)

    deg_r = _deg_kernel(dst3, ew3)
    deg_c = deg_r.reshape(B, NP, 1)

    g1 = _tc_matmul1(deg_c, xp, W1)
    s1 = _spmm_kernel(g1, src3, dst3, ew3)
    g2 = _tc_matmul_mid(deg_c, s1, b1, W2)
    s2 = _spmm_kernel(g2, src3, dst3, ew3)
    g3 = _tc_matmul_mid(deg_c, s2, b2, W22)
    s3 = _spmm_kernel(g3, src3, dst3, ew3)

    g4 = _tc_matmul_mid(deg_c, s3, b22, W3)
    g4r = g4.reshape(B, 1, NP)
    s4 = _spmv_kernel(g4r, src3, dst3, ew3)
    out = _tc_final(deg_r.reshape(B, NR, 128), s4.reshape(B, NR, 128), b3)
    return out.reshape(B, NP)[:, 1:N - 1]


# speedup vs baseline: 44.4528x; 1.3278x over previous
import functools
import jax
import jax.numpy as jnp
from jax import lax
from jax.experimental import pallas as pl
from jax.experimental.pallas import tpu as pltpu
from jax.experimental.pallas import tpu_sc as plsc

B = 4
N = 10000
NP = 10240
NR = NP // 128
E = 320000
EP = 327680
ER = EP // 128
F_IN = 128
D = 64

NC = 2
NS = 16
EPT = EP // NS
NBLK = EPT // 2048

_mesh = plsc.VectorSubcoreMesh(core_axis_name="c", subcore_axis_name="s")


NRED = NP // 10


def _zero_flat(ref, nwords):
    def body(i, _):
        ref[pl.ds(16 * i, 16)] = jnp.zeros((16,), jnp.float32)
        return 0
    lax.fori_loop(0, nwords // 16, body, 0)


def _slab_reduce_out(acc, slab, tmp, red, sid, out_slice):
    pltpu.sync_copy(acc, slab.at[sid])
    plsc.subcore_barrier()

    @pl.when(sid < 10)
    def _():
        pltpu.sync_copy(slab.at[0, pl.ds(NRED * sid, NRED)], red)

        def addk(k, _):
            pltpu.sync_copy(slab.at[k, pl.ds(NRED * sid, NRED)], tmp)

            def vadd(i, _):
                red[pl.ds(16 * i, 16)] = (red[pl.ds(16 * i, 16)]
                                          + tmp[pl.ds(16 * i, 16)])
                return 0
            lax.fori_loop(0, NRED // 16, vadd, 0)
            return 0
        lax.fori_loop(1, NS, addk, 0)
        out_slice(sid, red)
    plsc.subcore_barrier()


@functools.partial(
    pl.kernel,
    out_type=jax.ShapeDtypeStruct((B, 1, NP), jnp.float32),
    mesh=_mesh,
    compiler_params=pltpu.CompilerParams(needs_layout_passes=False, use_tc_tiling_on_sc=False),
    scratch_types=[
        pltpu.VMEM((16, 128), jnp.int32),
        pltpu.VMEM((16, 128), jnp.float32),
        pltpu.VMEM((NP,), jnp.float32),
        pltpu.VMEM((NRED,), jnp.float32),
        pltpu.VMEM((NRED,), jnp.float32),
        pltpu.VMEM_SHARED((NS, NP), jnp.float32),
    ],
)
def _deg_kernel(dst_hbm, ew_hbm, deg_hbm, dbuf, wbuf, acc, tmp, red, slab):
    cid = lax.axis_index("c")
    sid = lax.axis_index("s")
    row0 = sid * (EPT // 128)

    def per_graph(gi, _):
        b = 2 * cid + gi
        _zero_flat(acc, NP)

        def per_block(k, _):
            pltpu.sync_copy(dst_hbm.at[b, pl.ds(row0 + 16 * k, 16)], dbuf)
            pltpu.sync_copy(ew_hbm.at[b, pl.ds(row0 + 16 * k, 16)], wbuf)

            def per_row(j, _):
                for u in range(8):
                    dvec = dbuf[j, pl.ds(16 * u, 16)]
                    wvec = wbuf[j, pl.ds(16 * u, 16)]
                    plsc.addupdate_scatter(acc, [dvec], wvec)
                return 0
            lax.fori_loop(0, 16, per_row, 0)
            return 0
        lax.fori_loop(0, NBLK, per_block, 0)

        def out_slice(t, red_ref):
            pltpu.sync_copy(red_ref, deg_hbm.at[b, 0, pl.ds(NRED * t, NRED)])
        _slab_reduce_out(acc, slab, tmp, red, sid, out_slice)
        return 0
    lax.fori_loop(0, B // NC, per_graph, 0)


@functools.partial(
    pl.kernel,
    out_type=jax.ShapeDtypeStruct((B, NP, D), jnp.float32),
    mesh=_mesh,
    compiler_params=pltpu.CompilerParams(needs_layout_passes=False, use_tc_tiling_on_sc=False),
    scratch_types=[
        pltpu.VMEM((2, 16, 128), jnp.int32),
        pltpu.VMEM((2, 16, 128), jnp.int32),
        pltpu.VMEM((2 * 2048,), jnp.float32),
        pltpu.VMEM((4, 128, D), jnp.float32),
        pltpu.VMEM((64, D), jnp.float32),
        pltpu.SemaphoreType.DMA((4,)),
        pltpu.SemaphoreType.DMA((4,)),
        pltpu.VMEM_SHARED((NP, D), jnp.float32),
    ],
)
def _spmm_kernel(g_hbm, src_hbm, dst_hbm, ew_hbm, s_hbm,
                 sbuf, dbuf, wbuf, rows, zbuf, gsem, ssem, out_sh):
    cid = lax.axis_index("c")
    sid = lax.axis_index("s")
    row0 = sid * (EPT // 128)
    nslice = NP // NS
    nd0 = sid * nslice

    def zrow(r, _):
        for u in range(D // 16):
            zbuf[r, pl.ds(16 * u, 16)] = jnp.zeros((16,), jnp.float32)
        return 0
    lax.fori_loop(0, 64, zrow, 0)

    def per_graph(gi, _):
        b = 2 * cid + gi
        def zcp(m, _):
            pltpu.sync_copy(zbuf, out_sh.at[pl.ds(nd0 + 64 * m, 64)])
            return 0
        lax.fori_loop(0, nslice // 64, zcp, 0)
        plsc.subcore_barrier()

        def per_quad(q, _):
            blk = q // 4
            qq = lax.rem(q, 4)
            slot = lax.rem(blk, 2)

            @pl.when(qq == 0)
            def _():
                pltpu.sync_copy(src_hbm.at[b, pl.ds(row0 + 16 * blk, 16)],
                                sbuf.at[slot])
                pltpu.sync_copy(dst_hbm.at[b, pl.ds(row0 + 16 * blk, 16)],
                                dbuf.at[slot])
                pltpu.sync_copy(ew_hbm.at[b, pl.ds(sid * EPT + 2048 * blk, 2048)],
                                wbuf.at[pl.ds(slot * 2048, 2048)])

            handles = []
            for t in range(4):
                @pl.when(q > 0)
                def _(t=t):
                    pltpu.make_async_copy(g_hbm.at[b, pl.ds(0, 128)],
                                          rows.at[t], ssem.at[t]).wait()
                handles.append(pltpu.async_copy(
                    g_hbm.at[b].at[sbuf.at[slot, 4 * qq + t]],
                    rows.at[t], gsem.at[t]))

            for t in range(4):
                handles[t].wait()
                wbase = slot * 2048 + (4 * qq + t) * 128

                def scale(ee, _, t=t, wbase=wbase):
                    for d in range(4):
                        e = 4 * ee + d
                        w = plsc.load_gather(
                            wbuf, [jnp.full((16,), wbase + e, jnp.int32)])
                        for u in range(D // 16):
                            rows[t, e, pl.ds(16 * u, 16)] = (
                                rows[t, e, pl.ds(16 * u, 16)] * w)
                    return 0
                lax.fori_loop(0, 32, scale, 0)
                pltpu.async_copy(rows.at[t], out_sh.at[dbuf.at[slot, 4 * qq + t]],
                                 ssem.at[t], add=True)
            return 0
        lax.fori_loop(0, NBLK * 4, per_quad, 0)

        for t in range(4):
            pltpu.make_async_copy(g_hbm.at[b, pl.ds(0, 128)],
                                  rows.at[t], ssem.at[t]).wait()
        plsc.subcore_barrier()
        pltpu.sync_copy(out_sh.at[pl.ds(nd0, nslice)],
                        s_hbm.at[b, pl.ds(nd0, nslice)])
        plsc.subcore_barrier()
        return 0
    lax.fori_loop(0, B // NC, per_graph, 0)


@functools.partial(
    pl.kernel,
    out_type=jax.ShapeDtypeStruct((B, 1, NP), jnp.float32),
    mesh=_mesh,
    compiler_params=pltpu.CompilerParams(needs_layout_passes=False, use_tc_tiling_on_sc=False),
    scratch_types=[
        pltpu.VMEM((16, 128), jnp.int32),
        pltpu.VMEM((16, 128), jnp.int32),
        pltpu.VMEM((16, 128), jnp.float32),
        pltpu.VMEM((NP,), jnp.float32),
        pltpu.VMEM((NP,), jnp.float32),
        pltpu.VMEM((NRED,), jnp.float32),
        pltpu.VMEM((NRED,), jnp.float32),
        pltpu.VMEM_SHARED((NS, NP), jnp.float32),
    ],
)
def _spmv_kernel(g_hbm, src_hbm, dst_hbm, ew_hbm, s_hbm,
                 sbuf, dbuf, wbuf, gvec, acc, tmp, red, slab):
    cid = lax.axis_index("c")
    sid = lax.axis_index("s")
    row0 = sid * (EPT // 128)

    def per_graph(gi, _):
        b = 2 * cid + gi
        pltpu.sync_copy(g_hbm.at[b, 0], gvec)
        _zero_flat(acc, NP)

        def per_block(k, _):
            pltpu.sync_copy(src_hbm.at[b, pl.ds(row0 + 16 * k, 16)], sbuf)
            pltpu.sync_copy(dst_hbm.at[b, pl.ds(row0 + 16 * k, 16)], dbuf)
            pltpu.sync_copy(ew_hbm.at[b, pl.ds(row0 + 16 * k, 16)], wbuf)

            def per_row(j, _):
                for u in range(8):
                    svec = sbuf[j, pl.ds(16 * u, 16)]
                    dvec = dbuf[j, pl.ds(16 * u, 16)]
                    wvec = wbuf[j, pl.ds(16 * u, 16)]
                    vals = plsc.load_gather(gvec, [svec])
                    plsc.addupdate_scatter(acc, [dvec], vals * wvec)
                return 0
            lax.fori_loop(0, 16, per_row, 0)
            return 0
        lax.fori_loop(0, NBLK, per_block, 0)

        def out_slice(t, red_ref):
            pltpu.sync_copy(red_ref, s_hbm.at[b, 0, pl.ds(NRED * t, NRED)])
        _slab_reduce_out(acc, slab, tmp, red, sid, out_slice)
        return 0
    lax.fori_loop(0, B // NC, per_graph, 0)


def _dis(deg):
    return jnp.where(deg > 0, lax.rsqrt(jnp.where(deg > 0, deg, 1.0)), 0.0)


def _k1_body(deg_ref, x_ref, w_ref, o_ref):
    dis = _dis(deg_ref[0])
    o_ref[0] = jnp.dot(x_ref[0], w_ref[...],
                       preferred_element_type=jnp.float32) * dis


def _kmid_body(deg_ref, s_ref, b_ref, w_ref, o_ref):
    dis = _dis(deg_ref[0])
    x = jnp.maximum(s_ref[0] * dis + b_ref[...], 0.0)
    o_ref[0] = jnp.dot(x, w_ref[...],
                       preferred_element_type=jnp.float32) * dis


def _kfinal_body(deg_ref, s_ref, b_ref, o_ref):
    deg = deg_ref[0]
    o = s_ref[0] * _dis(deg) + b_ref[...]
    idx = (lax.broadcasted_iota(jnp.int32, (NR, 128), 0) * 128
           + lax.broadcasted_iota(jnp.int32, (NR, 128), 1))
    mask = (idx >= 1) & (idx < N - 1)
    m = jnp.max(jnp.where(mask, o, -jnp.inf))
    ex = jnp.where(mask, jnp.exp(o - m), 0.0)
    o_ref[0] = ex / jnp.sum(ex)


_RB = 1280
_NRB = NP // _RB


def _tc_matmul1(deg_c, xp, W):
    return pl.pallas_call(
        _k1_body,
        grid=(B, _NRB),
        in_specs=[
            pl.BlockSpec((1, _RB, 1), lambda b, i: (b, i, 0)),
            pl.BlockSpec((1, _RB, F_IN), lambda b, i: (b, i, 0)),
            pl.BlockSpec((F_IN, D), lambda b, i: (0, 0)),
        ],
        out_specs=pl.BlockSpec((1, _RB, D), lambda b, i: (b, i, 0)),
        out_shape=jax.ShapeDtypeStruct((B, NP, D), jnp.float32),
    )(deg_c, xp, W)


def _tc_matmul_mid(deg_c, s, bias, W):
    wo = W.shape[1]
    return pl.pallas_call(
        _kmid_body,
        grid=(B, _NRB),
        in_specs=[
            pl.BlockSpec((1, _RB, 1), lambda b, i: (b, i, 0)),
            pl.BlockSpec((1, _RB, D), lambda b, i: (b, i, 0)),
            pl.BlockSpec((D,), lambda b, i: (0,)),
            pl.BlockSpec((D, wo), lambda b, i: (0, 0)),
        ],
        out_specs=pl.BlockSpec((1, _RB, wo), lambda b, i: (b, i, 0)),
        out_shape=jax.ShapeDtypeStruct((B, NP, wo), jnp.float32),
    )(deg_c, s, bias, W)


def _tc_final(deg_r, s4, b3):
    return pl.pallas_call(
        _kfinal_body,
        grid=(B,),
        in_specs=[
            pl.BlockSpec((1, NR, 128), lambda b: (b, 0, 0)),
            pl.BlockSpec((1, NR, 128), lambda b: (b, 0, 0)),
            pl.BlockSpec((1,), lambda b: (0,)),
        ],
        out_specs=pl.BlockSpec((1, NR, 128), lambda b: (b, 0, 0)),
        out_shape=jax.ShapeDtypeStruct((B, NR, 128), jnp.float32),
    )(deg_r, s4, b3)


def kernel(batch_feat, batch_edges, batch_attr, W1, b1, W2, b2, W22, b22, W3, b3):
    xp = jnp.pad(batch_feat, ((0, 0), (0, NP - N), (0, 0)))
    src = jnp.pad(batch_edges[:, 0, :], ((0, 0), (0, EP - E)))
    dst = jnp.pad(batch_edges[:, 1, :], ((0, 0), (0, EP - E)))
    ew = jnp.pad(batch_attr, ((0, 0), (0, EP - E)))
    src3 = src.reshape(B, ER, 128)
    dst3 = dst.reshape(B, ER, 128)
    ew3 = ew.reshape(B, ER, 128)

    deg_r = _deg_kernel(dst3, ew3)
    deg_c = deg_r.reshape(B, NP, 1)

    g1 = _tc_matmul1(deg_c, xp, W1)
    s1 = _spmm_kernel(g1, src3, dst3, ew)
    g2 = _tc_matmul_mid(deg_c, s1, b1, W2)
    s2 = _spmm_kernel(g2, src3, dst3, ew)
    g3 = _tc_matmul_mid(deg_c, s2, b2, W22)
    s3 = _spmm_kernel(g3, src3, dst3, ew)

    g4 = _tc_matmul_mid(deg_c, s3, b22, W3)
    g4r = g4.reshape(B, 1, NP)
    s4 = _spmv_kernel(g4r, src3, dst3, ew3)
    out = _tc_final(deg_r.reshape(B, NR, 128), s4.reshape(B, NR, 128), b3)
    return out.reshape(B, NP)[:, 1:N - 1]
